# baseline (speedup 1.0000x reference)
import functools

import jax
import jax.numpy as jnp
from jax import lax
from jax.experimental import pallas as pl
from jax.experimental.pallas import tpu as pltpu
from jax.experimental.pallas import tpu_sc as plsc

NC = 2
NS = 16
NW = NC * NS
LANES = 16
SEQ_CHUNK = 2
WINDOWS = ((0, 80), (80, 80), (160, 40))

TBLK = 512


def _prep_kernel(tab_t_hbm, tail_hbm, out_hbm, src_v, dst_v, sem):
  emb, n_tok = tab_t_hbm.shape
  n_blocks = (n_tok // TBLK)
  n_tail = n_tok - n_blocks * TBLK
  wid = lax.axis_index("s") * NC + lax.axis_index("c")

  lane = lax.iota(jnp.int32, LANES)

  @pl.loop(wid, n_blocks, step=NW)
  def _block(cid):
    t0 = cid * TBLK
    pltpu.sync_copy(tab_t_hbm.at[:, pl.ds(t0, TBLK)], src_v)

    @pl.loop(0, emb)
    def _col(e):
      col = jnp.broadcast_to(e, (LANES,)).astype(jnp.int32)

      @pl.loop(0, TBLK, step=LANES)
      def _row(t):
        vals = src_v[e, pl.ds(t, LANES)]
        plsc.store_scatter(dst_v, [t + lane, col], vals)

    pltpu.sync_copy(dst_v, out_hbm.at[pl.ds(t0, TBLK)])

  @pl.when(wid == 0)
  def _tail():
    pltpu.sync_copy(tail_hbm, dst_v.at[pl.ds(0, n_tail)])
    pltpu.sync_copy(dst_v.at[pl.ds(0, n_tail)],
                    out_hbm.at[pl.ds(n_blocks * TBLK, n_tail)])


def _emb_kernel(tok_hbm, comb_hbm, idx_hbm, cidx_hbm, out_hbm,
                idx_v, cidx_v, tok_v, sem):
  n_seq, seq, _ = out_hbm.shape
  seq_per_tile = n_seq // NW
  rows_chunk = SEQ_CHUNK * seq
  wid = lax.axis_index("s") * NC + lax.axis_index("c")
  seq0 = wid * seq_per_tile

  @pl.loop(0, seq_per_tile, step=SEQ_CHUNK)
  def _chunk(soff):
    sbase = seq0 + soff
    rbase = sbase * seq
    pltpu.sync_copy(idx_hbm.at[pl.ds(rbase, rows_chunk)], idx_v)
    pltpu.sync_copy(cidx_hbm.at[pl.ds(rbase, rows_chunk)], cidx_v)

    copies = []
    for s in range(SEQ_CHUNK):
      for off, ln in WINDOWS:
        copies.append(pltpu.async_copy(
            tok_hbm.at[idx_v.at[pl.ds(s * seq + off, ln)]],
            tok_v.at[s, pl.ds(off, ln)], sem))
    for c in copies:
      c.wait()
    copies = []
    for s in range(SEQ_CHUNK):
      for off, ln in WINDOWS:
        copies.append(pltpu.async_copy(
            comb_hbm.at[cidx_v.at[pl.ds(s * seq + off, ln)]],
            tok_v.at[s, pl.ds(off, ln)], sem, add=True))
    for c in copies:
      c.wait()

    pltpu.sync_copy(tok_v, out_hbm.at[pl.ds(sbase, SEQ_CHUNK)])


def kernel(x, segment_label, token_table, position_table, segment_table):
  batch, seq = x.shape
  vocab, emb = token_table.shape
  n = batch * seq
  mesh = plsc.VectorSubcoreMesh(core_axis_name="c", subcore_axis_name="s",
                                num_cores=NC, num_subcores=NS)

  n_blocks = vocab // TBLK
  n_tail = vocab - n_blocks * TBLK
  tab_t = token_table.T
  tail128 = jnp.pad(token_table[n_blocks * TBLK:],
                    ((0, 0), (0, 128 - emb)))
  prep = pl.kernel(
      _prep_kernel,
      out_type=jax.ShapeDtypeStruct((vocab, 128), jnp.float32),
      mesh=mesh,
      scratch_types=[
          pltpu.VMEM((emb, TBLK), jnp.float32),
          pltpu.VMEM((TBLK, 128), jnp.float32),
          pltpu.SemaphoreType.DMA,
      ],
      compiler_params=pltpu.CompilerParams(use_tc_tiling_on_sc=True,
                                           needs_layout_passes=False),
  )
  table128 = prep(tab_t, tail128)

  nseg = segment_table.shape[0]
  combined = (position_table[:seq, None, :]
              + segment_table[None, :, :]).reshape(seq * nseg, emb)
  comb128 = jnp.pad(combined, ((0, 0), (0, 128 - emb)))

  idx = x.reshape(n).astype(jnp.int32)
  cidx = (jnp.arange(seq, dtype=jnp.int32)[None, :] * nseg
          + segment_label.astype(jnp.int32)).reshape(n)

  run = pl.kernel(
      _emb_kernel,
      out_type=jax.ShapeDtypeStruct((batch, seq, 128), jnp.float32),
      mesh=mesh,
      scratch_types=[
          pltpu.VMEM((SEQ_CHUNK * seq,), jnp.int32),
          pltpu.VMEM((SEQ_CHUNK * seq,), jnp.int32),
          pltpu.VMEM((SEQ_CHUNK, seq, 128), jnp.float32),
          pltpu.SemaphoreType.DMA,
      ],
      compiler_params=pltpu.CompilerParams(use_tc_tiling_on_sc=True),
  )
  out128 = run(table128, comb128, idx, cidx)
  return out128[:, :, :emb]


# speedup vs baseline: 1.0018x; 1.0018x over previous
import functools

import jax
import jax.numpy as jnp
from jax import lax
from jax.experimental import pallas as pl
from jax.experimental.pallas import tpu as pltpu
from jax.experimental.pallas import tpu_sc as plsc

NC = 2
NS = 16
NW = NC * NS
LANES = 16
SEQ_CHUNK = 2
WINDOWS = ((0, 80), (80, 80), (160, 40))

TBLK = 512


def _prep_kernel(tab_t_hbm, tail_hbm, out_hbm, src_v, dst_v, sem):
  emb, n_tok = tab_t_hbm.shape
  n_blocks = (n_tok // TBLK)
  n_tail = n_tok - n_blocks * TBLK
  wid = lax.axis_index("s") * NC + lax.axis_index("c")

  lane = lax.iota(jnp.int32, LANES)

  @pl.loop(wid, n_blocks, step=NW)
  def _block(cid):
    t0 = cid * TBLK
    pltpu.sync_copy(tab_t_hbm.at[:, pl.ds(t0, TBLK)], src_v)

    @pl.loop(0, emb)
    def _col(e):
      col = jnp.broadcast_to(e, (LANES,)).astype(jnp.int32)
      for g in range(TBLK // LANES):
        vals = src_v[e, pl.ds(g * LANES, LANES)]
        plsc.store_scatter(dst_v, [g * LANES + lane, col], vals)

    pltpu.sync_copy(dst_v, out_hbm.at[pl.ds(t0, TBLK)])

  @pl.when(wid == 0)
  def _tail():
    pltpu.sync_copy(tail_hbm, dst_v.at[pl.ds(0, n_tail)])
    pltpu.sync_copy(dst_v.at[pl.ds(0, n_tail)],
                    out_hbm.at[pl.ds(n_blocks * TBLK, n_tail)])


def _emb_kernel(tok_hbm, comb_hbm, idx_hbm, cidx_hbm, out_hbm,
                idx_v, cidx_v, tok_v, sem):
  n_seq, seq, _ = out_hbm.shape
  seq_per_tile = n_seq // NW
  rows_chunk = SEQ_CHUNK * seq
  wid = lax.axis_index("s") * NC + lax.axis_index("c")
  seq0 = wid * seq_per_tile

  @pl.loop(0, seq_per_tile, step=SEQ_CHUNK)
  def _chunk(soff):
    sbase = seq0 + soff
    rbase = sbase * seq
    pltpu.sync_copy(idx_hbm.at[pl.ds(rbase, rows_chunk)], idx_v)
    pltpu.sync_copy(cidx_hbm.at[pl.ds(rbase, rows_chunk)], cidx_v)

    copies = []
    for s in range(SEQ_CHUNK):
      for off, ln in WINDOWS:
        copies.append(pltpu.async_copy(
            tok_hbm.at[idx_v.at[pl.ds(s * seq + off, ln)]],
            tok_v.at[s, pl.ds(off, ln)], sem))
    for c in copies:
      c.wait()
    copies = []
    for s in range(SEQ_CHUNK):
      for off, ln in WINDOWS:
        copies.append(pltpu.async_copy(
            comb_hbm.at[cidx_v.at[pl.ds(s * seq + off, ln)]],
            tok_v.at[s, pl.ds(off, ln)], sem, add=True))
    for c in copies:
      c.wait()

    pltpu.sync_copy(tok_v, out_hbm.at[pl.ds(sbase, SEQ_CHUNK)])


def kernel(x, segment_label, token_table, position_table, segment_table):
  batch, seq = x.shape
  vocab, emb = token_table.shape
  n = batch * seq
  mesh = plsc.VectorSubcoreMesh(core_axis_name="c", subcore_axis_name="s",
                                num_cores=NC, num_subcores=NS)

  n_blocks = vocab // TBLK
  n_tail = vocab - n_blocks * TBLK
  tab_t = token_table.T
  tail128 = jnp.pad(token_table[n_blocks * TBLK:],
                    ((0, 0), (0, 128 - emb)))
  prep = pl.kernel(
      _prep_kernel,
      out_type=jax.ShapeDtypeStruct((vocab, 128), jnp.float32),
      mesh=mesh,
      scratch_types=[
          pltpu.VMEM((emb, TBLK), jnp.float32),
          pltpu.VMEM((TBLK, 128), jnp.float32),
          pltpu.SemaphoreType.DMA,
      ],
      compiler_params=pltpu.CompilerParams(use_tc_tiling_on_sc=True,
                                           needs_layout_passes=False),
  )
  table128 = prep(tab_t, tail128)

  nseg = segment_table.shape[0]
  combined = (position_table[:seq, None, :]
              + segment_table[None, :, :]).reshape(seq * nseg, emb)
  comb128 = jnp.pad(combined, ((0, 0), (0, 128 - emb)))

  idx = x.reshape(n).astype(jnp.int32)
  cidx = (jnp.arange(seq, dtype=jnp.int32)[None, :] * nseg
          + segment_label.astype(jnp.int32)).reshape(n)

  run = pl.kernel(
      _emb_kernel,
      out_type=jax.ShapeDtypeStruct((batch, seq, 128), jnp.float32),
      mesh=mesh,
      scratch_types=[
          pltpu.VMEM((SEQ_CHUNK * seq,), jnp.int32),
          pltpu.VMEM((SEQ_CHUNK * seq,), jnp.int32),
          pltpu.VMEM((SEQ_CHUNK, seq, 128), jnp.float32),
          pltpu.SemaphoreType.DMA,
      ],
      compiler_params=pltpu.CompilerParams(use_tc_tiling_on_sc=True),
  )
  out128 = run(table128, comb128, idx, cidx)
  return out128[:, :, :emb]
